# baseline (speedup 1.0000x reference)
import functools

import jax
import jax.numpy as jnp
from jax import lax
from jax.experimental import pallas as pl
from jax.experimental.pallas import tpu as pltpu
from jax.experimental.pallas import tpu_sc as plsc

N = 10000
E = 320000
D = 128
NS = 16
EPT = 20480
E_PAD = NS * EPT
RPT = 632
N_PAD = NS * RPT

A_CHUNK = 64
A_G = 16
A_NGROUP = EPT // (A_G * A_CHUNK)
NBUF = 5

_f32 = jnp.float32


def _wait_idx(e_ref, s, g, buf, sem):
    pltpu.make_async_copy(e_ref.at[s, pl.ds(g * A_G, A_G)], buf, sem).wait()


def _prefetch_idx(e_ref, s, g, buf, sem):
    pltpu.async_copy(e_ref.at[s, pl.ds(g * A_G, A_G)], buf, sem)


def _cnt_loop(s, dst_e, didx, ones_v, acc_s, ssems, isem):
    _prefetch_idx(dst_e, s, 0, didx.at[0], isem)

    def group_body(g, carry):
        par = lax.rem(g, 2)
        dg = didx.at[par]
        _wait_idx(dst_e, s, g, dg, isem)
        _prefetch_idx(dst_e, s, lax.min(g + 1, A_NGROUP - 1),
                      didx.at[1 - par], isem)
        sd = [
            pltpu.async_copy(ones_v, acc_s.at[dg.at[jj]],
                             ssems[jj % NBUF], add=True)
            for jj in range(A_G)
        ]
        for d in sd:
            d.wait()
        return carry

    lax.fori_loop(0, A_NGROUP, group_body, 0)
    _wait_idx(dst_e, s, 0, didx.at[A_NGROUP % 2], isem)


def _agg_loop(s, x_src, src_e, dst_e, sidx, didx, rowbufs, acc_s,
              gsems, ssems, isem0, isem1):
    _prefetch_idx(src_e, s, 0, sidx.at[0], isem0)
    _prefetch_idx(dst_e, s, 0, didx.at[0], isem1)

    def group_body(g, carry):
        par = lax.rem(g, 2)
        sg = sidx.at[par]
        dg = didx.at[par]
        _wait_idx(src_e, s, g, sg, isem0)
        _wait_idx(dst_e, s, g, dg, isem1)
        gn = lax.min(g + 1, A_NGROUP - 1)
        _prefetch_idx(src_e, s, gn, sidx.at[1 - par], isem0)
        _prefetch_idx(dst_e, s, gn, didx.at[1 - par], isem1)

        gd = [None] * A_G
        sd = [None] * A_G
        for jj in range(NBUF - 1):
            gd[jj] = pltpu.async_copy(
                x_src.at[sg.at[jj]], rowbufs[jj % NBUF], gsems[jj % NBUF])
        for jj in range(A_G):
            p = jj % NBUF
            gd[jj].wait()
            sd[jj] = pltpu.async_copy(
                rowbufs[p], acc_s.at[dg.at[jj]], ssems[p], add=True)
            nxt = jj + NBUF - 1
            if nxt < A_G:
                q = nxt % NBUF
                if nxt - NBUF >= 0:
                    sd[nxt - NBUF].wait()
                gd[nxt] = pltpu.async_copy(
                    x_src.at[sg.at[nxt]], rowbufs[q], gsems[q])
        for k in range(max(0, A_G - NBUF), A_G):
            sd[k].wait()
        return carry

    lax.fori_loop(0, A_NGROUP, group_body, 0)
    _wait_idx(src_e, s, 0, sidx.at[A_NGROUP % 2], isem0)
    _wait_idx(dst_e, s, 0, didx.at[A_NGROUP % 2], isem1)


def _split_scratch(rest):
    rowbufs = rest[:NBUF]
    acc_s = rest[NBUF]
    gsems = rest[NBUF + 1:2 * NBUF + 1]
    ssems = rest[2 * NBUF + 1:3 * NBUF + 1]
    isem0 = rest[3 * NBUF + 1]
    isem1 = rest[3 * NBUF + 2]
    return rowbufs, acc_s, gsems, ssems, isem0, isem1


def _cnt_agg_body(xu, xi, sui, dui, siu, diu, zrow, ones_h,
                  agg_i, agg_u, cnt_i, cnt_u,
                  sidx, didx, *rest):
    rowbufs, acc_s, gsems, ssems, isem0, isem1 = _split_scratch(rest)
    c = lax.axis_index("c")
    s = lax.axis_index("s")
    share = pl.ds(s * RPT, RPT)

    pltpu.sync_copy(zrow, acc_s.at[share])
    pltpu.sync_copy(ones_h, rowbufs[0])
    plsc.subcore_barrier()

    def do_cnt(dst_e, cnt_out):
        _cnt_loop(s, dst_e, didx, rowbufs[0], acc_s, ssems, isem1)
        plsc.subcore_barrier()
        pltpu.sync_copy(acc_s.at[share], cnt_out.at[share])
        pltpu.sync_copy(zrow, acc_s.at[share])
        plsc.subcore_barrier()

    @pl.when(c == 0)
    def _():
        do_cnt(dui, cnt_i)

    @pl.when(c == 1)
    def _():
        do_cnt(diu, cnt_u)

    def do_agg(x_src, src_e, dst_e, agg_out):
        _agg_loop(s, x_src, src_e, dst_e, sidx, didx, rowbufs, acc_s,
                  gsems, ssems, isem0, isem1)
        plsc.subcore_barrier()
        pltpu.sync_copy(acc_s.at[share], agg_out.at[share])

    @pl.when(c == 0)
    def _():
        do_agg(xu, sui, dui, agg_i)

    @pl.when(c == 1)
    def _():
        do_agg(xi, siu, diu, agg_u)


def _agg_body(xu, xi, sui, dui, siu, diu, zrow,
              agg_i, agg_u,
              sidx, didx, *rest):
    rowbufs, acc_s, gsems, ssems, isem0, isem1 = _split_scratch(rest)
    c = lax.axis_index("c")
    s = lax.axis_index("s")
    share = pl.ds(s * RPT, RPT)

    pltpu.sync_copy(zrow, acc_s.at[share])
    plsc.subcore_barrier()

    def do_agg(x_src, src_e, dst_e, agg_out):
        _agg_loop(s, x_src, src_e, dst_e, sidx, didx, rowbufs, acc_s,
                  gsems, ssems, isem0, isem1)
        plsc.subcore_barrier()
        pltpu.sync_copy(acc_s.at[share], agg_out.at[share])

    @pl.when(c == 0)
    def _():
        do_agg(xu, sui, dui, agg_i)

    @pl.when(c == 1)
    def _():
        do_agg(xi, siu, diu, agg_u)


def _sc_scratch():
    scratch = [
        pltpu.VMEM((2, A_G, A_CHUNK), jnp.int32),
        pltpu.VMEM((2, A_G, A_CHUNK), jnp.int32),
    ]
    scratch += [pltpu.VMEM((A_CHUNK, D), _f32) for _ in range(NBUF)]
    scratch.append(pltpu.VMEM_SHARED((N_PAD, D), _f32))
    scratch += [pltpu.SemaphoreType.DMA for _ in range(2 * NBUF + 2)]
    return scratch


def _make_cnt_agg():
    t = jax.ShapeDtypeStruct((N_PAD, D), _f32)
    return pl.kernel(
        _cnt_agg_body,
        out_type=(t, t, t, t),
        mesh=plsc.VectorSubcoreMesh(core_axis_name="c", subcore_axis_name="s"),
        scratch_types=_sc_scratch(),
        name="sage_cnt_agg",
    )


def _make_agg():
    t = jax.ShapeDtypeStruct((N_PAD, D), _f32)
    return pl.kernel(
        _agg_body,
        out_type=(t, t),
        mesh=plsc.VectorSubcoreMesh(core_axis_name="c", subcore_axis_name="s"),
        scratch_types=_sc_scratch(),
        name="sage_agg",
    )


_cnt_agg_pass = _make_cnt_agg()
_agg_pass = _make_agg()


def _mm_r_body(xi, Wri, bli, xu, Wru, blu, ri, ru):
    ri[...] = jnp.dot(xi[...], Wri[...], preferred_element_type=_f32) + bli[...]
    ru[...] = jnp.dot(xu[...], Wru[...], preferred_element_type=_f32) + blu[...]


_mm_r = pl.pallas_call(
    _mm_r_body,
    out_shape=(jax.ShapeDtypeStruct((N, D), _f32),
               jax.ShapeDtypeStruct((N, D), _f32)),
    name="sage_mm_r",
)


def _mm_l_body(relu, agg_i, cnt_i, ri, Wli, agg_u, cnt_u, ru, Wlu, hi, hu):
    def one(agg, cnt, r, Wl, out):
        deg = jnp.maximum(cnt[0:N, 0:1], 1.0)
        mean = agg[0:N, :] / deg
        h = jnp.dot(mean, Wl[...], preferred_element_type=_f32) + r[...]
        out[...] = jnp.maximum(h, 0.0) if relu else h

    one(agg_i, cnt_i, ri, Wli, hi)
    one(agg_u, cnt_u, ru, Wlu, hu)


def _make_mm_l(relu):
    return pl.pallas_call(
        functools.partial(_mm_l_body, relu),
        out_shape=(jax.ShapeDtypeStruct((N, D), _f32),
                   jax.ShapeDtypeStruct((N, D), _f32)),
        name="sage_mm_l_relu" if relu else "sage_mm_l",
    )


_mm_l_relu = _make_mm_l(True)
_mm_l_lin = _make_mm_l(False)


def _prep_edges(e):
    pad = E_PAD - E
    pad_src = jnp.arange(pad, dtype=jnp.int32) % N
    pad_dst = N + (jnp.arange(pad, dtype=jnp.int32) % (N_PAD - N))
    src = jnp.concatenate([e[0].astype(jnp.int32), pad_src])
    dst = jnp.concatenate([e[1].astype(jnp.int32), pad_dst])
    sa = src.reshape(NS, A_G * A_NGROUP, A_CHUNK)
    da = dst.reshape(NS, A_G * A_NGROUP, A_CHUNK)
    return sa, da


def kernel(x_user, x_item, edge_index_ui, edge_index_iu,
           W1_ui_l, b1_ui_l, W1_ui_r, W1_iu_l, b1_iu_l, W1_iu_r,
           W2_ui_l, b2_ui_l, W2_ui_r, W2_iu_l, b2_iu_l, W2_iu_r):
    sui, dui = _prep_edges(edge_index_ui)
    siu, diu = _prep_edges(edge_index_iu)
    zrow = jnp.zeros((RPT, D), _f32)
    ones_h = jnp.ones((A_CHUNK, D), _f32)

    r1_i, r1_u = _mm_r(
        x_item, W1_ui_r, b1_ui_l.reshape(1, D),
        x_user, W1_iu_r, b1_iu_l.reshape(1, D))

    agg1_i, agg1_u, cnt_i, cnt_u = _cnt_agg_pass(
        x_user, x_item, sui, dui, siu, diu, zrow, ones_h)
    h_item, h_user = _mm_l_relu(
        agg1_i, cnt_i, r1_i, W1_ui_l, agg1_u, cnt_u, r1_u, W1_iu_l)

    r2_i, r2_u = _mm_r(
        h_item, W2_ui_r, b2_ui_l.reshape(1, D),
        h_user, W2_iu_r, b2_iu_l.reshape(1, D))

    agg2_i, agg2_u = _agg_pass(h_user, h_item, sui, dui, siu, diu, zrow)
    out_item, out_user = _mm_l_lin(
        agg2_i, cnt_i, r2_i, W2_ui_l, agg2_u, cnt_u, r2_u, W2_iu_l)

    return (out_user, out_item)


# speedup vs baseline: 1.0093x; 1.0093x over previous
import functools

import jax
import jax.numpy as jnp
from jax import lax
from jax.experimental import pallas as pl
from jax.experimental.pallas import tpu as pltpu
from jax.experimental.pallas import tpu_sc as plsc

N = 10000
E = 320000
D = 128
NS = 16
EPT = 20480
E_PAD = NS * EPT
RPT = 632
N_PAD = NS * RPT

A_CHUNK = 64
A_G = 16
A_NGROUP = EPT // (A_G * A_CHUNK)
NBUF = 5

_f32 = jnp.float32


def _wait_idx(e_ref, s, g, buf, sem):
    pltpu.make_async_copy(e_ref.at[s, pl.ds(g * A_G, A_G)], buf, sem).wait()


def _prefetch_idx(e_ref, s, g, buf, sem):
    pltpu.async_copy(e_ref.at[s, pl.ds(g * A_G, A_G)], buf, sem)


def _cnt_loop(s, dst_e, didx, ones_v, acc_s, ssems, isem):
    _prefetch_idx(dst_e, s, 0, didx.at[0], isem)

    def group_body(g, carry):
        par = lax.rem(g, 2)
        dg = didx.at[par]
        _wait_idx(dst_e, s, g, dg, isem)
        _prefetch_idx(dst_e, s, lax.min(g + 1, A_NGROUP - 1),
                      didx.at[1 - par], isem)
        sd = [
            pltpu.async_copy(ones_v, acc_s.at[dg.at[jj]],
                             ssems[jj % NBUF], add=True)
            for jj in range(A_G)
        ]
        for d in sd:
            d.wait()
        return carry

    lax.fori_loop(0, A_NGROUP, group_body, 0)
    _wait_idx(dst_e, s, 0, didx.at[A_NGROUP % 2], isem)


def _agg_loop(s, x_src, src_e, dst_e, sidx, didx, rowbufs, acc_s,
              gsems, ssems, isem0, isem1):
    _prefetch_idx(src_e, s, 0, sidx.at[0], isem0)
    _prefetch_idx(dst_e, s, 0, didx.at[0], isem1)

    def group_body(g, carry):
        par = lax.rem(g, 2)
        sg = sidx.at[par]
        dg = didx.at[par]
        _wait_idx(src_e, s, g, sg, isem0)
        _wait_idx(dst_e, s, g, dg, isem1)
        gn = lax.min(g + 1, A_NGROUP - 1)
        _prefetch_idx(src_e, s, gn, sidx.at[1 - par], isem0)
        _prefetch_idx(dst_e, s, gn, didx.at[1 - par], isem1)

        gd = [None] * A_G
        sd = [None] * A_G
        for jj in range(NBUF - 1):
            gd[jj] = pltpu.async_copy(
                x_src.at[sg.at[jj]], rowbufs[jj % NBUF], gsems[jj % NBUF])
        for jj in range(A_G):
            p = jj % NBUF
            gd[jj].wait()
            sd[jj] = pltpu.async_copy(
                rowbufs[p], acc_s.at[dg.at[jj]], ssems[p], add=True)
            nxt = jj + NBUF - 1
            if nxt < A_G:
                q = nxt % NBUF
                if nxt - NBUF >= 0:
                    sd[nxt - NBUF].wait()
                gd[nxt] = pltpu.async_copy(
                    x_src.at[sg.at[nxt]], rowbufs[q], gsems[q])
        for k in range(max(0, A_G - NBUF), A_G):
            sd[k].wait()
        return carry

    lax.fori_loop(0, A_NGROUP, group_body, 0)
    _wait_idx(src_e, s, 0, sidx.at[A_NGROUP % 2], isem0)
    _wait_idx(dst_e, s, 0, didx.at[A_NGROUP % 2], isem1)


def _split_scratch(rest):
    rowbufs = rest[:NBUF]
    acc_s = rest[NBUF]
    gsems = rest[NBUF + 1:2 * NBUF + 1]
    ssems = rest[2 * NBUF + 1:3 * NBUF + 1]
    isem0 = rest[3 * NBUF + 1]
    isem1 = rest[3 * NBUF + 2]
    return rowbufs, acc_s, gsems, ssems, isem0, isem1


def _cnt_agg_body(xu, xi, sui, dui, siu, diu, zrow, ones_h,
                  agg_i, agg_u, cnt_i, cnt_u,
                  sidx, didx, *rest):
    rowbufs, acc_s, gsems, ssems, isem0, isem1 = _split_scratch(rest)
    c = lax.axis_index("c")
    s = lax.axis_index("s")
    share = pl.ds(s * RPT, RPT)

    pltpu.sync_copy(zrow, acc_s.at[share])
    pltpu.sync_copy(ones_h, rowbufs[0])
    plsc.subcore_barrier()

    def do_cnt(dst_e, cnt_out):
        _cnt_loop(s, dst_e, didx, rowbufs[0], acc_s, ssems, isem1)
        plsc.subcore_barrier()
        pltpu.sync_copy(acc_s.at[share], cnt_out.at[share])
        pltpu.sync_copy(zrow, acc_s.at[share])
        plsc.subcore_barrier()

    @pl.when(c == 0)
    def _():
        do_cnt(dui, cnt_i)

    @pl.when(c == 1)
    def _():
        do_cnt(diu, cnt_u)

    def do_agg(x_src, src_e, dst_e, agg_out):
        _agg_loop(s, x_src, src_e, dst_e, sidx, didx, rowbufs, acc_s,
                  gsems, ssems, isem0, isem1)
        plsc.subcore_barrier()
        pltpu.sync_copy(acc_s.at[share], agg_out.at[share])

    @pl.when(c == 0)
    def _():
        do_agg(xu, sui, dui, agg_i)

    @pl.when(c == 1)
    def _():
        do_agg(xi, siu, diu, agg_u)


def _agg_body(xu, xi, sui, dui, siu, diu, zrow,
              agg_i, agg_u,
              sidx, didx, *rest):
    rowbufs, acc_s, gsems, ssems, isem0, isem1 = _split_scratch(rest)
    c = lax.axis_index("c")
    s = lax.axis_index("s")
    share = pl.ds(s * RPT, RPT)

    pltpu.sync_copy(zrow, acc_s.at[share])
    plsc.subcore_barrier()

    def do_agg(x_src, src_e, dst_e, agg_out):
        _agg_loop(s, x_src, src_e, dst_e, sidx, didx, rowbufs, acc_s,
                  gsems, ssems, isem0, isem1)
        plsc.subcore_barrier()
        pltpu.sync_copy(acc_s.at[share], agg_out.at[share])

    @pl.when(c == 0)
    def _():
        do_agg(xu, sui, dui, agg_i)

    @pl.when(c == 1)
    def _():
        do_agg(xi, siu, diu, agg_u)


def _sc_scratch():
    scratch = [
        pltpu.VMEM((2, A_G, A_CHUNK), jnp.int32),
        pltpu.VMEM((2, A_G, A_CHUNK), jnp.int32),
    ]
    scratch += [pltpu.VMEM((A_CHUNK, D), _f32) for _ in range(NBUF)]
    scratch.append(pltpu.VMEM_SHARED((N_PAD, D), _f32))
    scratch += [pltpu.SemaphoreType.DMA for _ in range(2 * NBUF + 2)]
    return scratch


def _make_cnt_agg():
    t = jax.ShapeDtypeStruct((N_PAD, D), _f32)
    return pl.kernel(
        _cnt_agg_body,
        out_type=(t, t, t, t),
        mesh=plsc.VectorSubcoreMesh(core_axis_name="c", subcore_axis_name="s"),
        scratch_types=_sc_scratch(),
        name="sage_cnt_agg",
    )


def _make_agg():
    t = jax.ShapeDtypeStruct((N_PAD, D), _f32)
    return pl.kernel(
        _agg_body,
        out_type=(t, t),
        mesh=plsc.VectorSubcoreMesh(core_axis_name="c", subcore_axis_name="s"),
        scratch_types=_sc_scratch(),
        name="sage_agg",
    )


_cnt_agg_pass = _make_cnt_agg()
_agg_pass = _make_agg()


def _sage_mm_body(relu, agg_i, cnt_i, xi, Wli, bli, Wri,
                  agg_u, cnt_u, xu, Wlu, blu, Wru, hi, hu):
    def one(agg, cnt, x, Wl, b, Wr, out):
        deg = jnp.maximum(cnt[0:N, 0:1], 1.0)
        mean = agg[0:N, :] / deg
        h = (jnp.dot(mean, Wl[...], preferred_element_type=_f32)
             + b[...]
             + jnp.dot(x[...], Wr[...], preferred_element_type=_f32))
        out[...] = jnp.maximum(h, 0.0) if relu else h

    one(agg_i, cnt_i, xi, Wli, bli, Wri, hi)
    one(agg_u, cnt_u, xu, Wlu, blu, Wru, hu)


def _make_mm(relu):
    return pl.pallas_call(
        functools.partial(_sage_mm_body, relu),
        out_shape=(jax.ShapeDtypeStruct((N, D), _f32),
                   jax.ShapeDtypeStruct((N, D), _f32)),
        name="sage_mm_relu" if relu else "sage_mm",
    )


_mm_relu = _make_mm(True)
_mm_lin = _make_mm(False)


def _prep_edges(e):
    pad = E_PAD - E
    pad_src = jnp.arange(pad, dtype=jnp.int32) % N
    pad_dst = N + (jnp.arange(pad, dtype=jnp.int32) % (N_PAD - N))
    src = jnp.concatenate([e[0].astype(jnp.int32), pad_src])
    dst = jnp.concatenate([e[1].astype(jnp.int32), pad_dst])
    sa = src.reshape(NS, A_G * A_NGROUP, A_CHUNK)
    da = dst.reshape(NS, A_G * A_NGROUP, A_CHUNK)
    return sa, da


def kernel(x_user, x_item, edge_index_ui, edge_index_iu,
           W1_ui_l, b1_ui_l, W1_ui_r, W1_iu_l, b1_iu_l, W1_iu_r,
           W2_ui_l, b2_ui_l, W2_ui_r, W2_iu_l, b2_iu_l, W2_iu_r):
    sui, dui = _prep_edges(edge_index_ui)
    siu, diu = _prep_edges(edge_index_iu)
    zrow = jnp.zeros((RPT, D), _f32)
    ones_h = jnp.ones((A_CHUNK, D), _f32)

    agg1_i, agg1_u, cnt_i, cnt_u = _cnt_agg_pass(
        x_user, x_item, sui, dui, siu, diu, zrow, ones_h)
    h_item, h_user = _mm_relu(
        agg1_i, cnt_i, x_item, W1_ui_l, b1_ui_l.reshape(1, D), W1_ui_r,
        agg1_u, cnt_u, x_user, W1_iu_l, b1_iu_l.reshape(1, D), W1_iu_r)

    agg2_i, agg2_u = _agg_pass(h_user, h_item, sui, dui, siu, diu, zrow)
    out_item, out_user = _mm_lin(
        agg2_i, cnt_i, h_item, W2_ui_l, b2_ui_l.reshape(1, D), W2_ui_r,
        agg2_u, cnt_u, h_user, W2_iu_l, b2_iu_l.reshape(1, D), W2_iu_r)

    return (out_user, out_item)
